# baseline (speedup 1.0000x reference)
import functools

import jax
import jax.numpy as jnp
from jax import lax
from jax.experimental import pallas as pl
from jax.experimental.pallas import tpu as pltpu
from jax.experimental.pallas import tpu_sc as plsc

N = 10000
E = 320000
NP = N + 8
NC, NS = 2, 16
NW = NC * NS
CH = 128
EPW = 10240
NCHUNK = EPW // CH
E_PAD = NW * EPW
NCA = 2 * NCHUNK
HD2 = 64
NBUF = 4
RPS = 624

BN = 1000

def _mesh():
    return plsc.VectorSubcoreMesh(
        core_axis_name="c", subcore_axis_name="s",
        num_cores=NC, num_subcores=NS)


def _gru_body(up_ref, wih_ref, whh_ref, bih_ref, bhh_ref, gcnw_ref, degt_ref,
              xws_ref, dinv_ref):
    wih = wih_ref[...]
    whh = whh_ref[...]
    bih = bih_ref[...]
    bhh = bhh_ref[...]
    hd = whh.shape[0]
    h = jnp.zeros((up_ref.shape[1], hd), jnp.float32)
    for t in range(up_ref.shape[0]):
        x = up_ref[t]
        gi = jnp.dot(x, wih, preferred_element_type=jnp.float32) + bih
        gh = jnp.dot(h, whh, preferred_element_type=jnp.float32) + bhh
        r = jax.nn.sigmoid(gi[:, :hd] + gh[:, :hd])
        z = jax.nn.sigmoid(gi[:, hd:2 * hd] + gh[:, hd:2 * hd])
        n = jnp.tanh(gi[:, 2 * hd:] + r * gh[:, 2 * hd:])
        h = (1.0 - z) * n + z * h
    xw = jnp.dot(h, gcnw_ref[...], preferred_element_type=jnp.float32)
    deg = degt_ref[:, 0:1] + degt_ref[:, 1:2] + 1.0
    dinv = lax.rsqrt(deg)
    dinv_ref[...] = dinv
    xws = (xw * dinv).astype(jnp.bfloat16)
    xws_ref[0] = xws[:, :HD2]
    xws_ref[1] = xws[:, HD2:]


def _gru_call(up, wihT, whhT, bih, bhh, gcn_w, degt):
    seq, n, ind = up.shape
    g3 = wihT.shape[1]
    return pl.pallas_call(
        _gru_body,
        grid=(n // BN,),
        in_specs=[
            pl.BlockSpec((seq, BN, ind), lambda i: (0, i, 0)),
            pl.BlockSpec((ind, g3), lambda i: (0, 0)),
            pl.BlockSpec((whhT.shape[0], g3), lambda i: (0, 0)),
            pl.BlockSpec((1, g3), lambda i: (0, 0)),
            pl.BlockSpec((1, g3), lambda i: (0, 0)),
            pl.BlockSpec(gcn_w.shape, lambda i: (0, 0)),
            pl.BlockSpec((BN, 2), lambda i: (i, 0)),
        ],
        out_specs=[
            pl.BlockSpec((2, BN, HD2), lambda i: (0, i, 0)),
            pl.BlockSpec((BN, 1), lambda i: (i, 0)),
        ],
        out_shape=[
            jax.ShapeDtypeStruct((2, n, HD2), jnp.bfloat16),
            jax.ShapeDtypeStruct((n, 1), jnp.float32),
        ],
    )(up, wihT, whhT, bih, bhh, gcn_w, degt)


def _deg_body(dst_hbm, ones_hbm, zdeg_hbm, out_hbm, dst_v, ones_v, acc):
    c = lax.axis_index("c")
    s = lax.axis_index("s")
    wid = s * NC + c
    pltpu.sync_copy(dst_hbm.at[wid], dst_v)
    pltpu.sync_copy(ones_hbm, ones_v)

    @pl.when(s == 0)
    def _():
        pltpu.sync_copy(zdeg_hbm, acc)

    plsc.subcore_barrier()

    def body(g, carry):
        pltpu.sync_copy(ones_v, acc.at[dst_v.at[g]], add=True)
        return carry

    lax.fori_loop(0, NCHUNK, body, 0)
    plsc.subcore_barrier()

    @pl.when(s == 0)
    def _():
        pltpu.sync_copy(acc, out_hbm.at[c])


@functools.cache
def _deg_kernel_fn():
    return pl.kernel(
        _deg_body,
        out_type=jax.ShapeDtypeStruct((NC, NP), jnp.float32),
        mesh=_mesh(),
        scratch_types=[
            pltpu.VMEM((NCHUNK, CH), jnp.int32),
            pltpu.VMEM((CH,), jnp.float32),
            pltpu.VMEM_SHARED((NP,), jnp.float32),
        ],
        name="sc_degree",
    )


def _deg_kernel(dst_p, ones, zdeg):
    return _deg_kernel_fn()(dst_p, ones, zdeg)


def _agg_body(src_hbm, dst_hbm, xws_hbm, zrows_hbm, out_hbm,
              src_v, dst_v, rows0, rows1, rows2, rows3,
              gsem0, gsem1, gsem2, gsem3, ssem0, ssem1, ssem2, ssem3, acc):
    c = lax.axis_index("c")
    s = lax.axis_index("s")
    pltpu.sync_copy(src_hbm.at[s], src_v)
    pltpu.sync_copy(dst_hbm.at[s], dst_v)

    base = s * RPS
    tail = NS * RPS

    pltpu.sync_copy(xws_hbm.at[c].at[pl.ds(base, RPS)],
                    acc.at[pl.ds(base, RPS)])

    @pl.when(s == 0)
    def _():
        pltpu.sync_copy(xws_hbm.at[c].at[pl.ds(tail, N - tail)],
                        acc.at[pl.ds(tail, N - tail)])
        pltpu.sync_copy(zrows_hbm.at[pl.ds(0, NP - N)],
                        acc.at[pl.ds(N, NP - N)])

    plsc.subcore_barrier()

    table = xws_hbm.at[c]
    rows = (rows0, rows1, rows2, rows3)
    gsem = (gsem0, gsem1, gsem2, gsem3)
    ssem = (ssem0, ssem1, ssem2, ssem3)

    for b in range(NBUF):
        pltpu.async_copy(table.at[src_v.at[b]], rows[b], gsem[b])

    def body(i0, carry):
        for b in range(NBUF):
            j = i0 * NBUF + b
            pltpu.make_async_copy(table.at[src_v.at[j]], rows[b],
                                  gsem[b]).wait()
            pltpu.async_copy(rows[b], acc.at[dst_v.at[j]], ssem[b], add=True)
            kb = (b - 1) % NBUF
            k = j - 1

            @pl.when((k >= 0) & (k + NBUF < NCA))
            def _():
                pltpu.make_async_copy(rows[kb], acc.at[dst_v.at[k]],
                                      ssem[kb]).wait()
                pltpu.async_copy(table.at[src_v.at[k + NBUF]], rows[kb],
                                 gsem[kb])
        return carry

    lax.fori_loop(0, NCA // NBUF, body, 0)
    for b in range(NBUF):
        j = NCA - NBUF + b
        pltpu.make_async_copy(rows[b], acc.at[dst_v.at[j]], ssem[b]).wait()
    plsc.subcore_barrier()
    pltpu.sync_copy(acc.at[pl.ds(base, RPS)], out_hbm.at[c].at[pl.ds(base, RPS)])

    @pl.when(s == 0)
    def _():
        pltpu.sync_copy(acc.at[pl.ds(tail, N - tail)],
                        out_hbm.at[c].at[pl.ds(tail, N - tail)])


@functools.cache
def _agg_kernel_fn():
    return pl.kernel(
        _agg_body,
        out_type=jax.ShapeDtypeStruct((NC, NP, HD2), jnp.bfloat16),
        mesh=_mesh(),
        scratch_types=[
            pltpu.VMEM((NCA, CH), jnp.int32),
            pltpu.VMEM((NCA, CH), jnp.int32),
            pltpu.VMEM((CH, HD2), jnp.bfloat16),
            pltpu.VMEM((CH, HD2), jnp.bfloat16),
            pltpu.VMEM((CH, HD2), jnp.bfloat16),
            pltpu.VMEM((CH, HD2), jnp.bfloat16),
            pltpu.SemaphoreType.DMA,
            pltpu.SemaphoreType.DMA,
            pltpu.SemaphoreType.DMA,
            pltpu.SemaphoreType.DMA,
            pltpu.SemaphoreType.DMA,
            pltpu.SemaphoreType.DMA,
            pltpu.SemaphoreType.DMA,
            pltpu.SemaphoreType.DMA,
            pltpu.VMEM_SHARED((NP, HD2), jnp.bfloat16),
        ],
        compiler_params=pltpu.CompilerParams(use_tc_tiling_on_sc=False),
        name="sc_edge_aggregate",
    )


def _agg_kernel(src_p, dst_p, xws, zrows):
    return _agg_kernel_fn()(src_p, dst_p, xws, zrows)


def _heads_body(a0_ref, a1_ref, dinv_ref, gcnb_ref, f1w_ref, f1b_ref,
                f2w_ref, f2b_ref, clw_ref, clb_ref, md_ref, cl_ref):
    t = (jnp.concatenate([a0_ref[0], a1_ref[0]], axis=1).astype(jnp.float32)
         * dinv_ref[...] + gcnb_ref[...])
    h1 = jnp.maximum(
        jnp.dot(t, f1w_ref[...], preferred_element_type=jnp.float32)
        + f1b_ref[...], 0.0)
    md_ref[...] = (jnp.dot(h1, f2w_ref[...], preferred_element_type=jnp.float32)
                   + f2b_ref[...])
    cl_ref[...] = (jnp.dot(t, clw_ref[...], preferred_element_type=jnp.float32)
                   + clb_ref[...])


def _heads_call(accs, dinv, gcnb, f1w, f1b, f2w, f2b, clw, clb):
    d = 2 * accs.shape[2]
    ff = f1w.shape[1]
    k = clw.shape[1]
    return pl.pallas_call(
        _heads_body,
        grid=(N // BN,),
        in_specs=[
            pl.BlockSpec((1, BN, HD2), lambda i: (0, i, 0)),
            pl.BlockSpec((1, BN, HD2), lambda i: (1, i, 0)),
            pl.BlockSpec((BN, 1), lambda i: (i, 0)),
            pl.BlockSpec((1, d), lambda i: (0, 0)),
            pl.BlockSpec((d, ff), lambda i: (0, 0)),
            pl.BlockSpec((1, ff), lambda i: (0, 0)),
            pl.BlockSpec((ff, 1), lambda i: (0, 0)),
            pl.BlockSpec((1, 1), lambda i: (0, 0)),
            pl.BlockSpec((d, k), lambda i: (0, 0)),
            pl.BlockSpec((1, k), lambda i: (0, 0)),
        ],
        out_specs=[
            pl.BlockSpec((BN, 1), lambda i: (i, 0)),
            pl.BlockSpec((BN, k), lambda i: (i, 0)),
        ],
        out_shape=[
            jax.ShapeDtypeStruct((N, 1), jnp.float32),
            jax.ShapeDtypeStruct((N, k), jnp.float32),
        ],
    )(accs, accs, dinv, gcnb, f1w, f1b, f2w, f2b, clw, clb)


def kernel(user_profiles, interactions, edge_index, W_ih, W_hh, b_ih, b_hh,
           gcn_w, gcn_b, ff1_w, ff1_b, ff2_w, ff2_b, cl_w, cl_b):
    del interactions
    ei = edge_index.astype(jnp.int32)
    pad = E_PAD - E
    src_flat = jnp.concatenate([ei[0], jnp.zeros((pad,), jnp.int32)])
    dst_flat = jnp.concatenate([ei[1], jnp.full((pad,), N, jnp.int32)])
    dst_p = dst_flat.reshape(NW, NCHUNK, CH)
    src_a = src_flat.reshape(NS, NCA, CH)
    dst_a = dst_flat.reshape(NS, NCA, CH)
    ones = jnp.ones((CH,), jnp.float32)
    zdeg = jnp.zeros((NP,), jnp.float32)
    zrows = jnp.zeros((NP - N, HD2), jnp.bfloat16)

    degp = _deg_kernel(dst_p, ones, zdeg)
    degt = jnp.transpose(degp)[:N]
    xws, dinv = _gru_call(user_profiles, W_ih.T, W_hh.T,
                          b_ih.reshape(1, -1), b_hh.reshape(1, -1),
                          gcn_w, degt)
    accs = _agg_kernel(src_a, dst_a, xws, zrows)
    md, cl = _heads_call(accs, dinv, gcn_b.reshape(1, -1),
                         ff1_w, ff1_b.reshape(1, -1),
                         ff2_w, ff2_b.reshape(1, 1),
                         cl_w, cl_b.reshape(1, -1))
    return (md, cl)


# speedup vs baseline: 1.0287x; 1.0287x over previous
import functools

import jax
import jax.numpy as jnp
from jax import lax
from jax.experimental import pallas as pl
from jax.experimental.pallas import tpu as pltpu
from jax.experimental.pallas import tpu_sc as plsc

N = 10000
E = 320000
NP = N + 8
NC, NS = 2, 16
NW = NC * NS
CH = 128
EPW = 10240
NCHUNK = EPW // CH
E_PAD = NW * EPW
NCA = 2 * NCHUNK
HD2 = 64
NBUF = 4
RPS = 624

BN = 2000

def _mesh():
    return plsc.VectorSubcoreMesh(
        core_axis_name="c", subcore_axis_name="s",
        num_cores=NC, num_subcores=NS)


def _gru_body(up_ref, wih_ref, whh_ref, bih_ref, bhh_ref, gcnw_ref, degt_ref,
              xws_ref, dinv_ref):
    wih = wih_ref[...]
    whh = whh_ref[...]
    bih = bih_ref[...]
    bhh = bhh_ref[...]
    hd = whh.shape[0]
    h = jnp.zeros((up_ref.shape[1], hd), jnp.float32)
    for t in range(up_ref.shape[0]):
        x = up_ref[t].astype(jnp.bfloat16)
        gi = jnp.dot(x, wih, preferred_element_type=jnp.float32) + bih
        gh = jnp.dot(h.astype(jnp.bfloat16), whh,
                     preferred_element_type=jnp.float32) + bhh
        r = jax.nn.sigmoid(gi[:, :hd] + gh[:, :hd])
        z = jax.nn.sigmoid(gi[:, hd:2 * hd] + gh[:, hd:2 * hd])
        n = jnp.tanh(gi[:, 2 * hd:] + r * gh[:, 2 * hd:])
        h = (1.0 - z) * n + z * h
    xw = jnp.dot(h.astype(jnp.bfloat16), gcnw_ref[...],
                 preferred_element_type=jnp.float32)
    deg = degt_ref[:, 0:1] + degt_ref[:, 1:2] + 1.0
    dinv = lax.rsqrt(deg)
    dinv_ref[...] = dinv
    xws = (xw * dinv).astype(jnp.bfloat16)
    xws_ref[0] = xws[:, :HD2]
    xws_ref[1] = xws[:, HD2:]


def _gru_call(up, wihT, whhT, bih, bhh, gcn_w, degt):
    seq, n, ind = up.shape
    g3 = wihT.shape[1]
    return pl.pallas_call(
        _gru_body,
        grid=(n // BN,),
        in_specs=[
            pl.BlockSpec((seq, BN, ind), lambda i: (0, i, 0)),
            pl.BlockSpec((ind, g3), lambda i: (0, 0)),
            pl.BlockSpec((whhT.shape[0], g3), lambda i: (0, 0)),
            pl.BlockSpec((1, g3), lambda i: (0, 0)),
            pl.BlockSpec((1, g3), lambda i: (0, 0)),
            pl.BlockSpec(gcn_w.shape, lambda i: (0, 0)),
            pl.BlockSpec((BN, 2), lambda i: (i, 0)),
        ],
        out_specs=[
            pl.BlockSpec((2, BN, HD2), lambda i: (0, i, 0)),
            pl.BlockSpec((BN, 1), lambda i: (i, 0)),
        ],
        out_shape=[
            jax.ShapeDtypeStruct((2, n, HD2), jnp.bfloat16),
            jax.ShapeDtypeStruct((n, 1), jnp.float32),
        ],
    )(up, wihT, whhT, bih, bhh, gcn_w, degt)


def _deg_body(dst_hbm, ones_hbm, zdeg_hbm, out_hbm, dst_v, ones_v, acc):
    c = lax.axis_index("c")
    s = lax.axis_index("s")
    wid = s * NC + c
    pltpu.sync_copy(dst_hbm.at[wid], dst_v)
    pltpu.sync_copy(ones_hbm, ones_v)

    @pl.when(s == 0)
    def _():
        pltpu.sync_copy(zdeg_hbm, acc)

    plsc.subcore_barrier()

    def body(g, carry):
        pltpu.sync_copy(ones_v, acc.at[dst_v.at[g]], add=True)
        return carry

    lax.fori_loop(0, NCHUNK, body, 0)
    plsc.subcore_barrier()

    @pl.when(s == 0)
    def _():
        pltpu.sync_copy(acc, out_hbm.at[c])


@functools.cache
def _deg_kernel_fn():
    return pl.kernel(
        _deg_body,
        out_type=jax.ShapeDtypeStruct((NC, NP), jnp.float32),
        mesh=_mesh(),
        scratch_types=[
            pltpu.VMEM((NCHUNK, CH), jnp.int32),
            pltpu.VMEM((CH,), jnp.float32),
            pltpu.VMEM_SHARED((NP,), jnp.float32),
        ],
        name="sc_degree",
    )


def _deg_kernel(dst_p, ones, zdeg):
    return _deg_kernel_fn()(dst_p, ones, zdeg)


def _agg_body(src_hbm, dst_hbm, xws_hbm, zrows_hbm, out_hbm,
              src_v, dst_v, rows0, rows1, rows2, rows3,
              gsem0, gsem1, gsem2, gsem3, ssem0, ssem1, ssem2, ssem3, acc):
    c = lax.axis_index("c")
    s = lax.axis_index("s")
    pltpu.sync_copy(src_hbm.at[s], src_v)
    pltpu.sync_copy(dst_hbm.at[s], dst_v)

    base = s * RPS
    tail = NS * RPS

    pltpu.sync_copy(xws_hbm.at[c].at[pl.ds(base, RPS)],
                    acc.at[pl.ds(base, RPS)])

    @pl.when(s == 0)
    def _():
        pltpu.sync_copy(xws_hbm.at[c].at[pl.ds(tail, N - tail)],
                        acc.at[pl.ds(tail, N - tail)])
        pltpu.sync_copy(zrows_hbm.at[pl.ds(0, NP - N)],
                        acc.at[pl.ds(N, NP - N)])

    plsc.subcore_barrier()

    table = xws_hbm.at[c]
    rows = (rows0, rows1, rows2, rows3)
    gsem = (gsem0, gsem1, gsem2, gsem3)
    ssem = (ssem0, ssem1, ssem2, ssem3)

    for b in range(NBUF):
        pltpu.async_copy(table.at[src_v.at[b]], rows[b], gsem[b])

    def body(i0, carry):
        for b in range(NBUF):
            j = i0 * NBUF + b
            pltpu.make_async_copy(table.at[src_v.at[j]], rows[b],
                                  gsem[b]).wait()
            pltpu.async_copy(rows[b], acc.at[dst_v.at[j]], ssem[b], add=True)
            kb = (b - 1) % NBUF
            k = j - 1

            @pl.when((k >= 0) & (k + NBUF < NCA))
            def _():
                pltpu.make_async_copy(rows[kb], acc.at[dst_v.at[k]],
                                      ssem[kb]).wait()
                pltpu.async_copy(table.at[src_v.at[k + NBUF]], rows[kb],
                                 gsem[kb])
        return carry

    lax.fori_loop(0, NCA // NBUF, body, 0)
    for b in range(NBUF):
        j = NCA - NBUF + b
        pltpu.make_async_copy(rows[b], acc.at[dst_v.at[j]], ssem[b]).wait()
    plsc.subcore_barrier()
    pltpu.sync_copy(acc.at[pl.ds(base, RPS)], out_hbm.at[c].at[pl.ds(base, RPS)])

    @pl.when(s == 0)
    def _():
        pltpu.sync_copy(acc.at[pl.ds(tail, N - tail)],
                        out_hbm.at[c].at[pl.ds(tail, N - tail)])


@functools.cache
def _agg_kernel_fn():
    return pl.kernel(
        _agg_body,
        out_type=jax.ShapeDtypeStruct((NC, NP, HD2), jnp.bfloat16),
        mesh=_mesh(),
        scratch_types=[
            pltpu.VMEM((NCA, CH), jnp.int32),
            pltpu.VMEM((NCA, CH), jnp.int32),
            pltpu.VMEM((CH, HD2), jnp.bfloat16),
            pltpu.VMEM((CH, HD2), jnp.bfloat16),
            pltpu.VMEM((CH, HD2), jnp.bfloat16),
            pltpu.VMEM((CH, HD2), jnp.bfloat16),
            pltpu.SemaphoreType.DMA,
            pltpu.SemaphoreType.DMA,
            pltpu.SemaphoreType.DMA,
            pltpu.SemaphoreType.DMA,
            pltpu.SemaphoreType.DMA,
            pltpu.SemaphoreType.DMA,
            pltpu.SemaphoreType.DMA,
            pltpu.SemaphoreType.DMA,
            pltpu.VMEM_SHARED((NP, HD2), jnp.bfloat16),
        ],
        compiler_params=pltpu.CompilerParams(use_tc_tiling_on_sc=False),
        name="sc_edge_aggregate",
    )


def _agg_kernel(src_p, dst_p, xws, zrows):
    return _agg_kernel_fn()(src_p, dst_p, xws, zrows)


def _heads_body(a0_ref, a1_ref, dinv_ref, gcnb_ref, f1w_ref, f1b_ref,
                f2w_ref, f2b_ref, clw_ref, clb_ref, md_ref, cl_ref):
    t = (jnp.concatenate([a0_ref[0], a1_ref[0]], axis=1).astype(jnp.float32)
         * dinv_ref[...] + gcnb_ref[...])
    h1 = jnp.maximum(
        jnp.dot(t, f1w_ref[...], preferred_element_type=jnp.float32)
        + f1b_ref[...], 0.0)
    md_ref[...] = (jnp.dot(h1, f2w_ref[...], preferred_element_type=jnp.float32)
                   + f2b_ref[...])
    cl_ref[...] = (jnp.dot(t, clw_ref[...], preferred_element_type=jnp.float32)
                   + clb_ref[...])


def _heads_call(accs, dinv, gcnb, f1w, f1b, f2w, f2b, clw, clb):
    d = 2 * accs.shape[2]
    ff = f1w.shape[1]
    k = clw.shape[1]
    return pl.pallas_call(
        _heads_body,
        grid=(N // BN,),
        in_specs=[
            pl.BlockSpec((1, BN, HD2), lambda i: (0, i, 0)),
            pl.BlockSpec((1, BN, HD2), lambda i: (1, i, 0)),
            pl.BlockSpec((BN, 1), lambda i: (i, 0)),
            pl.BlockSpec((1, d), lambda i: (0, 0)),
            pl.BlockSpec((d, ff), lambda i: (0, 0)),
            pl.BlockSpec((1, ff), lambda i: (0, 0)),
            pl.BlockSpec((ff, 1), lambda i: (0, 0)),
            pl.BlockSpec((1, 1), lambda i: (0, 0)),
            pl.BlockSpec((d, k), lambda i: (0, 0)),
            pl.BlockSpec((1, k), lambda i: (0, 0)),
        ],
        out_specs=[
            pl.BlockSpec((BN, 1), lambda i: (i, 0)),
            pl.BlockSpec((BN, k), lambda i: (i, 0)),
        ],
        out_shape=[
            jax.ShapeDtypeStruct((N, 1), jnp.float32),
            jax.ShapeDtypeStruct((N, k), jnp.float32),
        ],
    )(accs, accs, dinv, gcnb, f1w, f1b, f2w, f2b, clw, clb)


def kernel(user_profiles, interactions, edge_index, W_ih, W_hh, b_ih, b_hh,
           gcn_w, gcn_b, ff1_w, ff1_b, ff2_w, ff2_b, cl_w, cl_b):
    del interactions
    ei = edge_index.astype(jnp.int32)
    pad = E_PAD - E
    src_flat = jnp.concatenate([ei[0], jnp.zeros((pad,), jnp.int32)])
    dst_flat = jnp.concatenate([ei[1], jnp.full((pad,), N, jnp.int32)])
    dst_p = dst_flat.reshape(NW, NCHUNK, CH)
    src_a = src_flat.reshape(NS, NCA, CH)
    dst_a = dst_flat.reshape(NS, NCA, CH)
    ones = jnp.ones((CH,), jnp.float32)
    zdeg = jnp.zeros((NP,), jnp.float32)
    zrows = jnp.zeros((NP - N, HD2), jnp.bfloat16)

    degp = _deg_kernel(dst_p, ones, zdeg)
    degt = jnp.transpose(degp)[:N]
    xws, dinv = _gru_call(user_profiles,
                          W_ih.T.astype(jnp.bfloat16),
                          W_hh.T.astype(jnp.bfloat16),
                          b_ih.reshape(1, -1), b_hh.reshape(1, -1),
                          gcn_w.astype(jnp.bfloat16), degt)
    accs = _agg_kernel(src_a, dst_a, xws, zrows)
    md, cl = _heads_call(accs, dinv, gcn_b.reshape(1, -1),
                         ff1_w, ff1_b.reshape(1, -1),
                         ff2_w, ff2_b.reshape(1, 1),
                         cl_w, cl_b.reshape(1, -1))
    return (md, cl)


# speedup vs baseline: 1.0375x; 1.0085x over previous
import functools

import jax
import jax.numpy as jnp
from jax import lax
from jax.experimental import pallas as pl
from jax.experimental.pallas import tpu as pltpu
from jax.experimental.pallas import tpu_sc as plsc

N = 10000
E = 320000
NP = N + 8
NC, NS = 2, 16
NW = NC * NS
CH = 128
EPW = 10240
NCHUNK = EPW // CH
E_PAD = NW * EPW
NCA = 2 * NCHUNK
HD2 = 64
NBUF = 4
RPS = 624

BN = 2000

def _mesh():
    return plsc.VectorSubcoreMesh(
        core_axis_name="c", subcore_axis_name="s",
        num_cores=NC, num_subcores=NS)


def _gru_body(up_ref, wih_ref, whh_ref, bih_ref, bhh_ref, gcnw_ref, degt_ref,
              xws_ref, dinv_ref):
    wih = wih_ref[...]
    whh = whh_ref[...]
    bih = bih_ref[...]
    bhh = bhh_ref[...]
    hd = whh.shape[0]
    h = jnp.zeros((up_ref.shape[1], hd), jnp.float32)
    for t in range(up_ref.shape[0]):
        x = up_ref[t]
        gi = jnp.dot(x, wih, preferred_element_type=jnp.float32) + bih
        gh = jnp.dot(h, whh, preferred_element_type=jnp.float32) + bhh
        r = jax.nn.sigmoid(gi[:, :hd] + gh[:, :hd])
        z = jax.nn.sigmoid(gi[:, hd:2 * hd] + gh[:, hd:2 * hd])
        n = jnp.tanh(gi[:, 2 * hd:] + r * gh[:, 2 * hd:])
        h = (1.0 - z) * n + z * h
    xw = jnp.dot(h, gcnw_ref[...], preferred_element_type=jnp.float32)
    deg = degt_ref[:, 0:1] + degt_ref[:, 1:2] + 1.0
    dinv = lax.rsqrt(deg)
    dinv_ref[...] = dinv
    xws = (xw * dinv).astype(jnp.bfloat16)
    xws_ref[0] = xws[:, :HD2]
    xws_ref[1] = xws[:, HD2:]


def _gru_call(up, wihT, whhT, bih, bhh, gcn_w, degt):
    seq, n, ind = up.shape
    g3 = wihT.shape[1]
    return pl.pallas_call(
        _gru_body,
        grid=(n // BN,),
        in_specs=[
            pl.BlockSpec((seq, BN, ind), lambda i: (0, i, 0)),
            pl.BlockSpec((ind, g3), lambda i: (0, 0)),
            pl.BlockSpec((whhT.shape[0], g3), lambda i: (0, 0)),
            pl.BlockSpec((1, g3), lambda i: (0, 0)),
            pl.BlockSpec((1, g3), lambda i: (0, 0)),
            pl.BlockSpec(gcn_w.shape, lambda i: (0, 0)),
            pl.BlockSpec((BN, 2), lambda i: (i, 0)),
        ],
        out_specs=[
            pl.BlockSpec((2, BN, HD2), lambda i: (0, i, 0)),
            pl.BlockSpec((BN, 1), lambda i: (i, 0)),
        ],
        out_shape=[
            jax.ShapeDtypeStruct((2, n, HD2), jnp.bfloat16),
            jax.ShapeDtypeStruct((n, 1), jnp.float32),
        ],
    )(up, wihT, whhT, bih, bhh, gcn_w, degt)


def _deg_body(dst_hbm, ones_hbm, zdeg_hbm, out_hbm, dst_v, ones_v, acc):
    c = lax.axis_index("c")
    s = lax.axis_index("s")
    wid = s * NC + c
    pltpu.sync_copy(dst_hbm.at[wid], dst_v)
    pltpu.sync_copy(ones_hbm, ones_v)

    @pl.when(s == 0)
    def _():
        pltpu.sync_copy(zdeg_hbm, acc)

    plsc.subcore_barrier()

    def body(g, carry):
        pltpu.sync_copy(ones_v, acc.at[dst_v.at[g]], add=True)
        return carry

    lax.fori_loop(0, NCHUNK, body, 0)
    plsc.subcore_barrier()

    @pl.when(s == 0)
    def _():
        pltpu.sync_copy(acc, out_hbm.at[c])


@functools.cache
def _deg_kernel_fn():
    return pl.kernel(
        _deg_body,
        out_type=jax.ShapeDtypeStruct((NC, NP), jnp.float32),
        mesh=_mesh(),
        scratch_types=[
            pltpu.VMEM((NCHUNK, CH), jnp.int32),
            pltpu.VMEM((CH,), jnp.float32),
            pltpu.VMEM_SHARED((NP,), jnp.float32),
        ],
        name="sc_degree",
    )


def _deg_kernel(dst_p, ones, zdeg):
    return _deg_kernel_fn()(dst_p, ones, zdeg)


def _agg_body(src_hbm, dst_hbm, xws_hbm, zrows_hbm, out_hbm,
              src_v, dst_v, rows0, rows1, rows2, rows3,
              gsem0, gsem1, gsem2, gsem3, ssem0, ssem1, ssem2, ssem3, acc):
    c = lax.axis_index("c")
    s = lax.axis_index("s")
    pltpu.sync_copy(src_hbm.at[s], src_v)
    pltpu.sync_copy(dst_hbm.at[s], dst_v)

    base = s * RPS
    tail = NS * RPS

    pltpu.sync_copy(xws_hbm.at[c].at[pl.ds(base, RPS)],
                    acc.at[pl.ds(base, RPS)])

    @pl.when(s == 0)
    def _():
        pltpu.sync_copy(xws_hbm.at[c].at[pl.ds(tail, N - tail)],
                        acc.at[pl.ds(tail, N - tail)])
        pltpu.sync_copy(zrows_hbm.at[pl.ds(0, NP - N)],
                        acc.at[pl.ds(N, NP - N)])

    plsc.subcore_barrier()

    table = xws_hbm.at[c]
    rows = (rows0, rows1, rows2, rows3)
    gsem = (gsem0, gsem1, gsem2, gsem3)
    ssem = (ssem0, ssem1, ssem2, ssem3)

    for b in range(NBUF):
        pltpu.async_copy(table.at[src_v.at[b]], rows[b], gsem[b])

    def body(i0, carry):
        for b in range(NBUF):
            j = i0 * NBUF + b
            pltpu.make_async_copy(table.at[src_v.at[j]], rows[b],
                                  gsem[b]).wait()
            pltpu.async_copy(rows[b], acc.at[dst_v.at[j]], ssem[b], add=True)
            kb = (b - 1) % NBUF
            k = j - 1

            @pl.when((k >= 0) & (k + NBUF < NCA))
            def _():
                pltpu.make_async_copy(rows[kb], acc.at[dst_v.at[k]],
                                      ssem[kb]).wait()
                pltpu.async_copy(table.at[src_v.at[k + NBUF]], rows[kb],
                                 gsem[kb])
        return carry

    lax.fori_loop(0, NCA // NBUF, body, 0)
    for b in range(NBUF):
        j = NCA - NBUF + b
        pltpu.make_async_copy(rows[b], acc.at[dst_v.at[j]], ssem[b]).wait()
    plsc.subcore_barrier()
    pltpu.sync_copy(acc.at[pl.ds(base, RPS)], out_hbm.at[c].at[pl.ds(base, RPS)])

    @pl.when(s == 0)
    def _():
        pltpu.sync_copy(acc.at[pl.ds(tail, N - tail)],
                        out_hbm.at[c].at[pl.ds(tail, N - tail)])


@functools.cache
def _agg_kernel_fn():
    return pl.kernel(
        _agg_body,
        out_type=jax.ShapeDtypeStruct((NC, NP, HD2), jnp.bfloat16),
        mesh=_mesh(),
        scratch_types=[
            pltpu.VMEM((NCA, CH), jnp.int32),
            pltpu.VMEM((NCA, CH), jnp.int32),
            pltpu.VMEM((CH, HD2), jnp.bfloat16),
            pltpu.VMEM((CH, HD2), jnp.bfloat16),
            pltpu.VMEM((CH, HD2), jnp.bfloat16),
            pltpu.VMEM((CH, HD2), jnp.bfloat16),
            pltpu.SemaphoreType.DMA,
            pltpu.SemaphoreType.DMA,
            pltpu.SemaphoreType.DMA,
            pltpu.SemaphoreType.DMA,
            pltpu.SemaphoreType.DMA,
            pltpu.SemaphoreType.DMA,
            pltpu.SemaphoreType.DMA,
            pltpu.SemaphoreType.DMA,
            pltpu.VMEM_SHARED((NP, HD2), jnp.bfloat16),
        ],
        compiler_params=pltpu.CompilerParams(use_tc_tiling_on_sc=False),
        name="sc_edge_aggregate",
    )


def _agg_kernel(src_p, dst_p, xws, zrows):
    return _agg_kernel_fn()(src_p, dst_p, xws, zrows)


def _heads_body(a0_ref, a1_ref, dinv_ref, gcnb_ref, f1w_ref, f1b_ref,
                f2w_ref, f2b_ref, clw_ref, clb_ref, md_ref, cl_ref):
    t = (jnp.concatenate([a0_ref[0], a1_ref[0]], axis=1).astype(jnp.float32)
         * dinv_ref[...] + gcnb_ref[...])
    h1 = jnp.maximum(
        jnp.dot(t, f1w_ref[...], preferred_element_type=jnp.float32)
        + f1b_ref[...], 0.0)
    md_ref[...] = (jnp.dot(h1, f2w_ref[...], preferred_element_type=jnp.float32)
                   + f2b_ref[...])
    cl_ref[...] = (jnp.dot(t, clw_ref[...], preferred_element_type=jnp.float32)
                   + clb_ref[...])


def _heads_call(accs, dinv, gcnb, f1w, f1b, f2w, f2b, clw, clb):
    d = 2 * accs.shape[2]
    ff = f1w.shape[1]
    k = clw.shape[1]
    return pl.pallas_call(
        _heads_body,
        grid=(N // BN,),
        in_specs=[
            pl.BlockSpec((1, BN, HD2), lambda i: (0, i, 0)),
            pl.BlockSpec((1, BN, HD2), lambda i: (1, i, 0)),
            pl.BlockSpec((BN, 1), lambda i: (i, 0)),
            pl.BlockSpec((1, d), lambda i: (0, 0)),
            pl.BlockSpec((d, ff), lambda i: (0, 0)),
            pl.BlockSpec((1, ff), lambda i: (0, 0)),
            pl.BlockSpec((ff, 1), lambda i: (0, 0)),
            pl.BlockSpec((1, 1), lambda i: (0, 0)),
            pl.BlockSpec((d, k), lambda i: (0, 0)),
            pl.BlockSpec((1, k), lambda i: (0, 0)),
        ],
        out_specs=[
            pl.BlockSpec((BN, 1), lambda i: (i, 0)),
            pl.BlockSpec((BN, k), lambda i: (i, 0)),
        ],
        out_shape=[
            jax.ShapeDtypeStruct((N, 1), jnp.float32),
            jax.ShapeDtypeStruct((N, k), jnp.float32),
        ],
    )(accs, accs, dinv, gcnb, f1w, f1b, f2w, f2b, clw, clb)


def kernel(user_profiles, interactions, edge_index, W_ih, W_hh, b_ih, b_hh,
           gcn_w, gcn_b, ff1_w, ff1_b, ff2_w, ff2_b, cl_w, cl_b):
    del interactions
    ei = edge_index.astype(jnp.int32)
    pad = E_PAD - E
    src_flat = jnp.concatenate([ei[0], jnp.zeros((pad,), jnp.int32)])
    dst_flat = jnp.concatenate([ei[1], jnp.full((pad,), N, jnp.int32)])
    dst_p = dst_flat.reshape(NW, NCHUNK, CH)
    src_a = src_flat.reshape(NS, NCA, CH)
    dst_a = dst_flat.reshape(NS, NCA, CH)
    ones = jnp.ones((CH,), jnp.float32)
    zdeg = jnp.zeros((NP,), jnp.float32)
    zrows = jnp.zeros((NP - N, HD2), jnp.bfloat16)

    degp = _deg_kernel(dst_p, ones, zdeg)
    degt = jnp.transpose(degp)[:N]
    xws, dinv = _gru_call(user_profiles, W_ih.T, W_hh.T,
                          b_ih.reshape(1, -1), b_hh.reshape(1, -1),
                          gcn_w, degt)
    accs = _agg_kernel(src_a, dst_a, xws, zrows)
    md, cl = _heads_call(accs, dinv, gcn_b.reshape(1, -1),
                         ff1_w, ff1_b.reshape(1, -1),
                         ff2_w, ff2_b.reshape(1, 1),
                         cl_w, cl_b.reshape(1, -1))
    return (md, cl)
